# baseline (speedup 1.0000x reference)
import jax
import jax.numpy as jnp
from jax import lax
from jax.experimental import pallas as pl
from jax.experimental.pallas import tpu as pltpu
from jax.experimental.pallas import tpu_sc as plsc

N = 100000
D = 128
LANES = 16
NC = 2
NS = 16
NW = NC * NS

CH = 128
NB = 3
NFULL = N // CH
TAIL = N - NFULL * CH
TAIL_WID = NFULL % NW
MAXCH = (NFULL + NW - 1) // NW
MINCH = NFULL // NW


def _body(ordering_hbm, noise_hbm, table_hbm, mean_hbm, lstd_hbm, out_hbm,
          idx_v, rows_v, noise_v, mean_v, lstd_v,
          gsem0, gsem1, gsem2, nsem0, nsem1, nsem2,
          isem0, isem1, isem2, osem0, osem1, osem2, msem):
    gsem = [gsem0, gsem1, gsem2]
    nsem = [nsem0, nsem1, nsem2]
    isem = [isem0, isem1, isem2]
    osem = [osem0, osem1, osem2]

    wid = lax.axis_index("s") * NC + lax.axis_index("c")
    n_w = (NFULL + NW - 1 - wid) // NW
    starts = [(wid + NW * c) * CH for c in range(MAXCH)]

    def exists(c):
        if c < MINCH:
            return True
        if c >= MAXCH:
            return False
        return c < n_w

    def run_if(cond, fn):
        if cond is True:
            fn()
        elif cond is not False:
            def wrapped():
                fn()
            pl.when(cond)(wrapped)

    pltpu.async_copy(mean_hbm, mean_v, msem)
    pltpu.async_copy(lstd_hbm, lstd_v, msem)

    def issue_fetch(c, buf):
        pltpu.async_copy(table_hbm.at[idx_v.at[buf]], rows_v.at[buf],
                         gsem[buf])
        pltpu.async_copy(noise_hbm.at[pl.ds(starts[c], CH)],
                         noise_v.at[buf], nsem[buf])

    def issue_idx(c, buf):
        pltpu.async_copy(ordering_hbm.at[pl.ds(starts[c], CH)],
                         idx_v.at[buf], isem[buf])

    def wait_fetch(buf):
        pltpu.make_async_copy(table_hbm.at[idx_v.at[buf]], rows_v.at[buf],
                              gsem[buf]).wait()
        pltpu.make_async_copy(noise_hbm.at[pl.ds(0, CH)], noise_v.at[buf],
                              nsem[buf]).wait()

    def wait_idx(buf):
        pltpu.make_async_copy(ordering_hbm.at[pl.ds(0, CH)], idx_v.at[buf],
                              isem[buf]).wait()

    def wait_out(buf):
        pltpu.make_async_copy(rows_v.at[buf], out_hbm.at[pl.ds(0, CH)],
                              osem[buf]).wait()

    issue_idx(0, 0)
    issue_idx(1, 1)
    wait_idx(0)
    issue_fetch(0, 0)
    wait_idx(1)
    issue_fetch(1, 1)
    issue_idx(2, 2)
    pltpu.make_async_copy(mean_hbm, mean_v, msem).wait()
    pltpu.make_async_copy(lstd_hbm, lstd_v, msem).wait()
    mean_vals = [mean_v[pl.ds(j * LANES, LANES)] for j in range(D // LANES)]
    scale_vals = [jnp.exp(lstd_v[pl.ds(j * LANES, LANES)])
                  for j in range(D // LANES)]

    def compute_rows(buf, nrows):
        def row_body(r, _):
            for j in range(D // LANES):
                sl = pl.ds(j * LANES, LANES)
                rows_v[buf, r, sl] = (rows_v[buf, r, sl] + mean_vals[j]
                                      + scale_vals[j] * noise_v[buf, r, sl])
            return None
        lax.fori_loop(0, nrows, row_body, None)

    for kk in range(MAXCH):
        b = kk % NB
        b2 = (kk + 2) % NB
        run_if(exists(kk), lambda: wait_fetch(b))
        run_if(exists(kk + 3), lambda: issue_idx(kk + 3, b))

        def produce(kk=kk, b=b):
            compute_rows(b, CH)
            pltpu.async_copy(rows_v.at[b],
                             out_hbm.at[pl.ds(starts[kk], CH)], osem[b])
        run_if(exists(kk), produce)

        run_if(exists(kk + 2), lambda: wait_idx(b2))
        if kk >= 1:
            run_if(exists(kk + 2), lambda: wait_out(b2))
        run_if(exists(kk + 2), lambda: issue_fetch(kk + 2, b2))

    wait_out(0)
    wait_out(1)
    wait_out(2)

    @pl.when(wid == TAIL_WID)
    def _tail():
        start = NFULL * CH
        pltpu.sync_copy(ordering_hbm.at[pl.ds(start, TAIL)],
                        idx_v.at[0, pl.ds(0, TAIL)])
        gather = pltpu.async_copy(table_hbm.at[idx_v.at[0, pl.ds(0, TAIL)]],
                                  rows_v.at[0, pl.ds(0, TAIL)], gsem[0])
        pltpu.sync_copy(noise_hbm.at[pl.ds(start, TAIL)],
                        noise_v.at[0, pl.ds(0, TAIL)])
        gather.wait()

        def row_body(r, _):
            for j in range(D // LANES):
                sl = pl.ds(j * LANES, LANES)
                rows_v[0, r, sl] = (rows_v[0, r, sl] + mean_vals[j]
                                    + scale_vals[j] * noise_v[0, r, sl])
            return None
        lax.fori_loop(0, TAIL, row_body, None)
        pltpu.sync_copy(rows_v.at[0, pl.ds(0, TAIL)],
                        out_hbm.at[pl.ds(start, TAIL)])


@jax.jit
def _run(ordering, noise, pos_embed, rand_mean, rand_log_std):
    mesh = plsc.VectorSubcoreMesh(core_axis_name="c", subcore_axis_name="s",
                                  num_cores=NC, num_subcores=NS)
    f = pl.kernel(
        _body,
        out_type=jax.ShapeDtypeStruct((N, D), jnp.float32),
        mesh=mesh,
        scratch_types=[
            pltpu.VMEM((NB, CH), jnp.int32),
            pltpu.VMEM((NB, CH, D), jnp.float32),
            pltpu.VMEM((NB, CH, D), jnp.float32),
            pltpu.VMEM((D,), jnp.float32),
            pltpu.VMEM((D,), jnp.float32),
        ] + [pltpu.SemaphoreType.DMA] * 13,
    )
    return f(ordering, noise, pos_embed, rand_mean, rand_log_std)


def kernel(ordering, noise, pos_embed, rand_mean, rand_log_std):
    return _run(ordering.astype(jnp.int32), noise, pos_embed,
                rand_mean, rand_log_std)


# speedup vs baseline: 1.0408x; 1.0408x over previous
import jax
import jax.numpy as jnp
from jax import lax
from jax.experimental import pallas as pl
from jax.experimental.pallas import tpu as pltpu
from jax.experimental.pallas import tpu_sc as plsc

N = 100000
D = 128
LANES = 16
NC = 2
NS = 16
NW = NC * NS

CH = 128
NB = 3
NT = 8
NCHUNK = NB * NT
BIG = 3128
SMALL = 3120
NBIG = 20
TAIL_BIG = BIG - NCHUNK * CH
TAIL_SMALL = SMALL - NCHUNK * CH


def _body(ordering_hbm, noise_hbm, table_hbm, mean_hbm, lstd_hbm, out_hbm,
          idx_all, rows_v, noise_v, mean_v, lstd_v,
          gsem0, gsem1, gsem2, nsem0, nsem1, nsem2,
          osem0, osem1, osem2, xsem, msem):
    gsem = [gsem0, gsem1, gsem2]
    nsem = [nsem0, nsem1, nsem2]
    osem = [osem0, osem1, osem2]

    wid = lax.axis_index("s") * NC + lax.axis_index("c")
    is_big = wid < NBIG
    base8 = 390 * wid + jnp.minimum(wid, NBIG)
    base_w = 8 * base8
    tail_start = base_w + NCHUNK * CH

    def chunk_start(c):
        return 8 * (base8 + (CH // 8) * c)

    pltpu.async_copy(ordering_hbm.at[pl.ds(base_w, SMALL)],
                     idx_all.at[pl.ds(0, SMALL)], xsem)

    @pl.when(is_big)
    def _():
        pltpu.async_copy(ordering_hbm.at[pl.ds(tail_start + TAIL_SMALL, 8)],
                         idx_all.at[pl.ds(SMALL, 8)], xsem)

    pltpu.async_copy(mean_hbm, mean_v, msem)
    pltpu.async_copy(lstd_hbm, lstd_v, msem)

    def issue_fetch(c, buf):
        pltpu.async_copy(
            table_hbm.at[idx_all.at[pl.ds(c * CH, CH)]], rows_v.at[buf],
            gsem[buf])
        pltpu.async_copy(noise_hbm.at[pl.ds(chunk_start(c), CH)],
                         noise_v.at[buf], nsem[buf])

    def wait_fetch(buf):
        pltpu.make_async_copy(table_hbm.at[idx_all.at[pl.ds(0, CH)]],
                              rows_v.at[buf], gsem[buf]).wait()
        pltpu.make_async_copy(noise_hbm.at[pl.ds(0, CH)], noise_v.at[buf],
                              nsem[buf]).wait()

    def wait_out(buf):
        pltpu.make_async_copy(rows_v.at[buf], out_hbm.at[pl.ds(0, CH)],
                              osem[buf]).wait()

    pltpu.make_async_copy(ordering_hbm.at[pl.ds(0, SMALL)],
                          idx_all.at[pl.ds(0, SMALL)], xsem).wait()

    @pl.when(is_big)
    def _():
        pltpu.make_async_copy(ordering_hbm.at[pl.ds(0, 8)],
                              idx_all.at[pl.ds(SMALL, 8)], xsem).wait()

    issue_fetch(0, 0)
    issue_fetch(1, 1)
    pltpu.make_async_copy(mean_hbm, mean_v, msem).wait()
    pltpu.make_async_copy(lstd_hbm, lstd_v, msem).wait()
    mean_vals = [mean_v[pl.ds(j * LANES, LANES)] for j in range(D // LANES)]
    scale_vals = [jnp.exp(lstd_v[pl.ds(j * LANES, LANES)])
                  for j in range(D // LANES)]

    def compute_rows(buf, nrows):
        def row_body(r, _):
            for j in range(D // LANES):
                sl = pl.ds(j * LANES, LANES)
                rows_v[buf, r, sl] = (rows_v[buf, r, sl] + mean_vals[j]
                                      + scale_vals[j] * noise_v[buf, r, sl])
            return None
        lax.fori_loop(0, nrows, row_body, None)

    @pl.loop(0, NT)
    def _triple(t):
        c0 = NB * t
        for i in range(NB):
            b = i
            b2 = (i + 2) % NB
            c = c0 + i
            wait_fetch(b)
            compute_rows(b, CH)
            pltpu.async_copy(rows_v.at[b],
                             out_hbm.at[pl.ds(chunk_start(c), CH)], osem[b])
            if i == 0:
                @pl.when(t > 0)
                def _():
                    wait_out(b2)
                issue_fetch(c + 2, b2)
            else:
                @pl.when(t < NT - 1)
                def _():
                    wait_out(b2)
                    issue_fetch(c + 2, b2)

    wait_out(0)

    def tail(nrows):
        toff = NCHUNK * CH
        pltpu.async_copy(
            table_hbm.at[idx_all.at[pl.ds(toff, nrows)]],
            rows_v.at[0, pl.ds(0, nrows)], gsem[0])
        pltpu.async_copy(noise_hbm.at[pl.ds(tail_start, nrows)],
                         noise_v.at[0, pl.ds(0, nrows)], nsem[0])
        pltpu.make_async_copy(table_hbm.at[idx_all.at[pl.ds(0, nrows)]],
                              rows_v.at[0, pl.ds(0, nrows)], gsem[0]).wait()
        pltpu.make_async_copy(noise_hbm.at[pl.ds(0, nrows)],
                              noise_v.at[0, pl.ds(0, nrows)], nsem[0]).wait()
        compute_rows(0, nrows)
        pltpu.async_copy(rows_v.at[0, pl.ds(0, nrows)],
                         out_hbm.at[pl.ds(tail_start, nrows)], osem[0])
        pltpu.make_async_copy(rows_v.at[0, pl.ds(0, nrows)],
                              out_hbm.at[pl.ds(0, nrows)], osem[0]).wait()

    @pl.when(is_big)
    def _():
        tail(TAIL_BIG)

    @pl.when(jnp.logical_not(is_big))
    def _():
        tail(TAIL_SMALL)

    wait_out(1)
    wait_out(2)


@jax.jit
def _run(ordering, noise, pos_embed, rand_mean, rand_log_std):
    mesh = plsc.VectorSubcoreMesh(core_axis_name="c", subcore_axis_name="s",
                                  num_cores=NC, num_subcores=NS)
    f = pl.kernel(
        _body,
        out_type=jax.ShapeDtypeStruct((N, D), jnp.float32),
        mesh=mesh,
        scratch_types=[
            pltpu.VMEM((BIG,), jnp.int32),
            pltpu.VMEM((NB, CH, D), jnp.float32),
            pltpu.VMEM((NB, CH, D), jnp.float32),
            pltpu.VMEM((D,), jnp.float32),
            pltpu.VMEM((D,), jnp.float32),
        ] + [pltpu.SemaphoreType.DMA] * 11,
    )
    return f(ordering, noise, pos_embed, rand_mean, rand_log_std)


def kernel(ordering, noise, pos_embed, rand_mean, rand_log_std):
    return _run(ordering.astype(jnp.int32), noise, pos_embed,
                rand_mean, rand_log_std)


# speedup vs baseline: 1.0468x; 1.0057x over previous
import jax
import jax.numpy as jnp
from jax import lax
from jax.experimental import pallas as pl
from jax.experimental.pallas import tpu as pltpu
from jax.experimental.pallas import tpu_sc as plsc

N = 100000
D = 128
LANES = 16
NC = 2
NS = 16
NW = NC * NS

CH = 128
NB = 3
NT = 8
NCHUNK = NB * NT
BIG = 3128
SMALL = 3120
NBIG = 20
TAIL_BIG = BIG - NCHUNK * CH
TAIL_SMALL = SMALL - NCHUNK * CH


def _body(ordering_hbm, noise_hbm, table_hbm, mean_hbm, lstd_hbm, out_hbm,
          idx_all, rows_v, noise_v, trows_v, tnoise_v, mean_v, lstd_v,
          gsem0, gsem1, gsem2, nsem0, nsem1, nsem2,
          osem0, osem1, osem2, xsem, msem, tsem, tosem):
    gsem = [gsem0, gsem1, gsem2]
    nsem = [nsem0, nsem1, nsem2]
    osem = [osem0, osem1, osem2]

    wid = lax.axis_index("s") * NC + lax.axis_index("c")
    is_big = wid < NBIG
    base8 = 390 * wid + jnp.minimum(wid, NBIG)
    base_w = 8 * base8
    tail_start = base_w + NCHUNK * CH

    def chunk_start(c):
        return 8 * (base8 + (CH // 8) * c)

    pltpu.async_copy(ordering_hbm.at[pl.ds(base_w, SMALL)],
                     idx_all.at[pl.ds(0, SMALL)], xsem)

    @pl.when(is_big)
    def _():
        pltpu.async_copy(ordering_hbm.at[pl.ds(tail_start + TAIL_SMALL, 8)],
                         idx_all.at[pl.ds(SMALL, 8)], xsem)

    pltpu.async_copy(noise_hbm.at[pl.ds(chunk_start(0), CH)],
                     noise_v.at[0], nsem[0])
    pltpu.async_copy(noise_hbm.at[pl.ds(chunk_start(1), CH)],
                     noise_v.at[1], nsem[1])
    pltpu.async_copy(mean_hbm, mean_v, msem)
    pltpu.async_copy(lstd_hbm, lstd_v, msem)

    @pl.when(is_big)
    def _():
        pltpu.async_copy(noise_hbm.at[pl.ds(tail_start, TAIL_BIG)],
                         tnoise_v.at[pl.ds(0, TAIL_BIG)], tsem)

    @pl.when(jnp.logical_not(is_big))
    def _():
        pltpu.async_copy(noise_hbm.at[pl.ds(tail_start, TAIL_SMALL)],
                         tnoise_v.at[pl.ds(0, TAIL_SMALL)], tsem)

    pltpu.make_async_copy(ordering_hbm.at[pl.ds(0, SMALL)],
                          idx_all.at[pl.ds(0, SMALL)], xsem).wait()

    @pl.when(is_big)
    def _():
        pltpu.make_async_copy(ordering_hbm.at[pl.ds(0, 8)],
                              idx_all.at[pl.ds(SMALL, 8)], xsem).wait()

    pltpu.async_copy(table_hbm.at[idx_all.at[pl.ds(0, CH)]], rows_v.at[0],
                     gsem[0])
    pltpu.async_copy(table_hbm.at[idx_all.at[pl.ds(CH, CH)]], rows_v.at[1],
                     gsem[1])
    TOFF = NCHUNK * CH

    @pl.when(is_big)
    def _():
        pltpu.async_copy(table_hbm.at[idx_all.at[pl.ds(TOFF, TAIL_BIG)]],
                         trows_v.at[pl.ds(0, TAIL_BIG)], tsem)

    @pl.when(jnp.logical_not(is_big))
    def _():
        pltpu.async_copy(table_hbm.at[idx_all.at[pl.ds(TOFF, TAIL_SMALL)]],
                         trows_v.at[pl.ds(0, TAIL_SMALL)], tsem)

    pltpu.make_async_copy(mean_hbm, mean_v, msem).wait()
    pltpu.make_async_copy(lstd_hbm, lstd_v, msem).wait()
    mean_vals = [mean_v[pl.ds(j * LANES, LANES)] for j in range(D // LANES)]
    scale_vals = [jnp.exp(lstd_v[pl.ds(j * LANES, LANES)])
                  for j in range(D // LANES)]

    def issue_fetch(c, buf):
        pltpu.async_copy(
            table_hbm.at[idx_all.at[pl.ds(c * CH, CH)]], rows_v.at[buf],
            gsem[buf])
        pltpu.async_copy(noise_hbm.at[pl.ds(chunk_start(c), CH)],
                         noise_v.at[buf], nsem[buf])

    def wait_fetch(buf):
        pltpu.make_async_copy(table_hbm.at[idx_all.at[pl.ds(0, CH)]],
                              rows_v.at[buf], gsem[buf]).wait()
        pltpu.make_async_copy(noise_hbm.at[pl.ds(0, CH)], noise_v.at[buf],
                              nsem[buf]).wait()

    def wait_out(buf):
        pltpu.make_async_copy(rows_v.at[buf], out_hbm.at[pl.ds(0, CH)],
                              osem[buf]).wait()

    def compute_rows(buf, nrows):
        def row_body(r, _):
            for j in range(D // LANES):
                sl = pl.ds(j * LANES, LANES)
                rows_v[buf, r, sl] = (rows_v[buf, r, sl] + mean_vals[j]
                                      + scale_vals[j] * noise_v[buf, r, sl])
            return None
        lax.fori_loop(0, nrows, row_body, None)

    @pl.loop(0, NT)
    def _triple(t):
        c0 = NB * t
        for i in range(NB):
            b = i
            b2 = (i + 2) % NB
            c = c0 + i
            wait_fetch(b)
            compute_rows(b, CH)
            pltpu.async_copy(rows_v.at[b],
                             out_hbm.at[pl.ds(chunk_start(c), CH)], osem[b])
            if i == 0:
                @pl.when(t > 0)
                def _():
                    wait_out(b2)
                issue_fetch(c + 2, b2)
            else:
                @pl.when(t < NT - 1)
                def _():
                    wait_out(b2)
                    issue_fetch(c + 2, b2)

    def tail(nrows):
        pltpu.make_async_copy(table_hbm.at[idx_all.at[pl.ds(0, nrows)]],
                              trows_v.at[pl.ds(0, nrows)], tsem).wait()
        pltpu.make_async_copy(noise_hbm.at[pl.ds(0, nrows)],
                              tnoise_v.at[pl.ds(0, nrows)], tsem).wait()

        def row_body(r, _):
            for j in range(D // LANES):
                sl = pl.ds(j * LANES, LANES)
                trows_v[r, sl] = (trows_v[r, sl] + mean_vals[j]
                                  + scale_vals[j] * tnoise_v[r, sl])
            return None
        lax.fori_loop(0, nrows, row_body, None)
        pltpu.async_copy(trows_v.at[pl.ds(0, nrows)],
                         out_hbm.at[pl.ds(tail_start, nrows)], tosem)
        pltpu.make_async_copy(trows_v.at[pl.ds(0, nrows)],
                              out_hbm.at[pl.ds(0, nrows)], tosem).wait()

    @pl.when(is_big)
    def _():
        tail(TAIL_BIG)

    @pl.when(jnp.logical_not(is_big))
    def _():
        tail(TAIL_SMALL)

    wait_out(0)
    wait_out(1)
    wait_out(2)


@jax.jit
def _run(ordering, noise, pos_embed, rand_mean, rand_log_std):
    mesh = plsc.VectorSubcoreMesh(core_axis_name="c", subcore_axis_name="s",
                                  num_cores=NC, num_subcores=NS)
    f = pl.kernel(
        _body,
        out_type=jax.ShapeDtypeStruct((N, D), jnp.float32),
        mesh=mesh,
        scratch_types=[
            pltpu.VMEM((BIG,), jnp.int32),
            pltpu.VMEM((NB, CH, D), jnp.float32),
            pltpu.VMEM((NB, CH, D), jnp.float32),
            pltpu.VMEM((TAIL_BIG, D), jnp.float32),
            pltpu.VMEM((TAIL_BIG, D), jnp.float32),
            pltpu.VMEM((D,), jnp.float32),
            pltpu.VMEM((D,), jnp.float32),
        ] + [pltpu.SemaphoreType.DMA] * 13,
    )
    return f(ordering, noise, pos_embed, rand_mean, rand_log_std)


def kernel(ordering, noise, pos_embed, rand_mean, rand_log_std):
    return _run(ordering.astype(jnp.int32), noise, pos_embed,
                rand_mean, rand_log_std)


# speedup vs baseline: 1.0501x; 1.0031x over previous
import jax
import jax.numpy as jnp
from jax import lax
from jax.experimental import pallas as pl
from jax.experimental.pallas import tpu as pltpu
from jax.experimental.pallas import tpu_sc as plsc

N = 100000
D = 128
LANES = 16
NC = 2
NS = 16
NW = NC * NS

CH = 128
NB = 3
NT = 8
NCHUNK = NB * NT
BIG = 3128
SMALL = 3120
NBIG = 20
TAIL_BIG = BIG - NCHUNK * CH
TAIL_SMALL = SMALL - NCHUNK * CH


def _body(ordering_hbm, noise_hbm, table_hbm, mean_hbm, lstd_hbm, out_hbm,
          idx_all, rows_v, noise_v, trows_v, tnoise_v, mean_v, lstd_v,
          gsem0, gsem1, gsem2, nsem0, nsem1, nsem2,
          osem0, osem1, osem2, xsem, msem, tsem, tosem):
    gsem = [gsem0, gsem1, gsem2]
    nsem = [nsem0, nsem1, nsem2]
    osem = [osem0, osem1, osem2]

    wid = lax.axis_index("s") * NC + lax.axis_index("c")
    is_big = wid < NBIG
    base8 = 390 * wid + jnp.minimum(wid, NBIG)
    base_w = 8 * base8
    tail_start = base_w + NCHUNK * CH

    def chunk_start(c):
        return 8 * (base8 + (CH // 8) * c)

    pltpu.async_copy(ordering_hbm.at[pl.ds(base_w, SMALL)],
                     idx_all.at[pl.ds(0, SMALL)], xsem)

    @pl.when(is_big)
    def _():
        pltpu.async_copy(ordering_hbm.at[pl.ds(tail_start + TAIL_SMALL, 8)],
                         idx_all.at[pl.ds(SMALL, 8)], xsem)

    pltpu.async_copy(noise_hbm.at[pl.ds(chunk_start(0), CH)],
                     noise_v.at[0], nsem[0])
    pltpu.async_copy(noise_hbm.at[pl.ds(chunk_start(1), CH)],
                     noise_v.at[1], nsem[1])
    pltpu.async_copy(mean_hbm, mean_v, msem)
    pltpu.async_copy(lstd_hbm, lstd_v, msem)

    @pl.when(is_big)
    def _():
        pltpu.async_copy(noise_hbm.at[pl.ds(tail_start, TAIL_BIG)],
                         tnoise_v.at[pl.ds(0, TAIL_BIG)], tsem)

    @pl.when(jnp.logical_not(is_big))
    def _():
        pltpu.async_copy(noise_hbm.at[pl.ds(tail_start, TAIL_SMALL)],
                         tnoise_v.at[pl.ds(0, TAIL_SMALL)], tsem)

    pltpu.make_async_copy(ordering_hbm.at[pl.ds(0, SMALL)],
                          idx_all.at[pl.ds(0, SMALL)], xsem).wait()

    @pl.when(is_big)
    def _():
        pltpu.make_async_copy(ordering_hbm.at[pl.ds(0, 8)],
                              idx_all.at[pl.ds(SMALL, 8)], xsem).wait()

    pltpu.async_copy(table_hbm.at[idx_all.at[pl.ds(0, CH)]], rows_v.at[0],
                     gsem[0])
    pltpu.async_copy(table_hbm.at[idx_all.at[pl.ds(CH, CH)]], rows_v.at[1],
                     gsem[1])
    TOFF = NCHUNK * CH

    @pl.when(is_big)
    def _():
        pltpu.async_copy(table_hbm.at[idx_all.at[pl.ds(TOFF, TAIL_BIG)]],
                         trows_v.at[pl.ds(0, TAIL_BIG)], tsem)

    @pl.when(jnp.logical_not(is_big))
    def _():
        pltpu.async_copy(table_hbm.at[idx_all.at[pl.ds(TOFF, TAIL_SMALL)]],
                         trows_v.at[pl.ds(0, TAIL_SMALL)], tsem)

    pltpu.make_async_copy(mean_hbm, mean_v, msem).wait()
    pltpu.make_async_copy(lstd_hbm, lstd_v, msem).wait()
    mean_vals = [mean_v[pl.ds(j * LANES, LANES)] for j in range(D // LANES)]
    scale_vals = [jnp.exp(lstd_v[pl.ds(j * LANES, LANES)])
                  for j in range(D // LANES)]

    def issue_fetch(c, buf):
        pltpu.async_copy(
            table_hbm.at[idx_all.at[pl.ds(c * CH, CH)]], rows_v.at[buf],
            gsem[buf])
        pltpu.async_copy(noise_hbm.at[pl.ds(chunk_start(c), CH)],
                         noise_v.at[buf], nsem[buf])

    def wait_fetch(buf):
        pltpu.make_async_copy(table_hbm.at[idx_all.at[pl.ds(0, CH)]],
                              rows_v.at[buf], gsem[buf]).wait()
        pltpu.make_async_copy(noise_hbm.at[pl.ds(0, CH)], noise_v.at[buf],
                              nsem[buf]).wait()

    def wait_out(buf):
        pltpu.make_async_copy(rows_v.at[buf], out_hbm.at[pl.ds(0, CH)],
                              osem[buf]).wait()

    def compute_rows(buf, nrows):
        def row_body(h, _):
            r = 2 * h
            for dr in range(2):
                for j in range(D // LANES):
                    sl = pl.ds(j * LANES, LANES)
                    rows_v[buf, r + dr, sl] = (
                        rows_v[buf, r + dr, sl] + mean_vals[j]
                        + scale_vals[j] * noise_v[buf, r + dr, sl])
            return None
        lax.fori_loop(0, nrows // 2, row_body, None)

    @pl.loop(0, NT)
    def _triple(t):
        c0 = NB * t
        for i in range(NB):
            b = i
            b2 = (i + 2) % NB
            c = c0 + i
            wait_fetch(b)
            compute_rows(b, CH)
            pltpu.async_copy(rows_v.at[b],
                             out_hbm.at[pl.ds(chunk_start(c), CH)], osem[b])
            if i == 0:
                @pl.when(t > 0)
                def _():
                    wait_out(b2)
                issue_fetch(c + 2, b2)
            else:
                @pl.when(t < NT - 1)
                def _():
                    wait_out(b2)
                    issue_fetch(c + 2, b2)

    def tail(nrows):
        pltpu.make_async_copy(table_hbm.at[idx_all.at[pl.ds(0, nrows)]],
                              trows_v.at[pl.ds(0, nrows)], tsem).wait()
        pltpu.make_async_copy(noise_hbm.at[pl.ds(0, nrows)],
                              tnoise_v.at[pl.ds(0, nrows)], tsem).wait()

        def row_body(r, _):
            for j in range(D // LANES):
                sl = pl.ds(j * LANES, LANES)
                trows_v[r, sl] = (trows_v[r, sl] + mean_vals[j]
                                  + scale_vals[j] * tnoise_v[r, sl])
            return None
        lax.fori_loop(0, nrows, row_body, None)
        pltpu.async_copy(trows_v.at[pl.ds(0, nrows)],
                         out_hbm.at[pl.ds(tail_start, nrows)], tosem)
        pltpu.make_async_copy(trows_v.at[pl.ds(0, nrows)],
                              out_hbm.at[pl.ds(0, nrows)], tosem).wait()

    @pl.when(is_big)
    def _():
        tail(TAIL_BIG)

    @pl.when(jnp.logical_not(is_big))
    def _():
        tail(TAIL_SMALL)

    wait_out(0)
    wait_out(1)
    wait_out(2)


@jax.jit
def _run(ordering, noise, pos_embed, rand_mean, rand_log_std):
    mesh = plsc.VectorSubcoreMesh(core_axis_name="c", subcore_axis_name="s",
                                  num_cores=NC, num_subcores=NS)
    f = pl.kernel(
        _body,
        out_type=jax.ShapeDtypeStruct((N, D), jnp.float32),
        mesh=mesh,
        scratch_types=[
            pltpu.VMEM((BIG,), jnp.int32),
            pltpu.VMEM((NB, CH, D), jnp.float32),
            pltpu.VMEM((NB, CH, D), jnp.float32),
            pltpu.VMEM((TAIL_BIG, D), jnp.float32),
            pltpu.VMEM((TAIL_BIG, D), jnp.float32),
            pltpu.VMEM((D,), jnp.float32),
            pltpu.VMEM((D,), jnp.float32),
        ] + [pltpu.SemaphoreType.DMA] * 13,
    )
    return f(ordering, noise, pos_embed, rand_mean, rand_log_std)


def kernel(ordering, noise, pos_embed, rand_mean, rand_log_std):
    return _run(ordering.astype(jnp.int32), noise, pos_embed,
                rand_mean, rand_log_std)


# speedup vs baseline: 1.0700x; 1.0190x over previous
import jax
import jax.numpy as jnp
from jax import lax
from jax.experimental import pallas as pl
from jax.experimental.pallas import tpu as pltpu
from jax.experimental.pallas import tpu_sc as plsc

N = 100000
D = 128
LANES = 16
NC = 2
NS = 16
NW = NC * NS

CH = 128
NB = 3
NT = 8
NCHUNK = NB * NT
BIG = 3128
SMALL = 3120
NBIG = 20
TAIL_BIG = BIG - NCHUNK * CH
TAIL_SMALL = SMALL - NCHUNK * CH


def _body(ordering_hbm, noise_hbm, table_hbm, mean_hbm, lstd_hbm, out_hbm,
          idx_all, idx0_v, rows_v, noise_v, trows_v, tnoise_v, mean_v, lstd_v,
          gsem0, gsem1, gsem2, nsem0, nsem1, nsem2,
          osem0, osem1, osem2, xsem, x0sem, msem, tsem, tosem):
    gsem = [gsem0, gsem1, gsem2]
    nsem = [nsem0, nsem1, nsem2]
    osem = [osem0, osem1, osem2]

    wid = lax.axis_index("s") * NC + lax.axis_index("c")
    is_big = wid < NBIG
    base8 = 390 * wid + jnp.minimum(wid, NBIG)
    base_w = 8 * base8
    tail_start = base_w + NCHUNK * CH

    def chunk_start(c):
        return 8 * (base8 + (CH // 8) * c)

    pltpu.async_copy(ordering_hbm.at[pl.ds(base_w, CH)], idx0_v, x0sem)
    pltpu.async_copy(ordering_hbm.at[pl.ds(base_w, SMALL)],
                     idx_all.at[pl.ds(0, SMALL)], xsem)

    @pl.when(is_big)
    def _():
        pltpu.async_copy(ordering_hbm.at[pl.ds(tail_start + TAIL_SMALL, 8)],
                         idx_all.at[pl.ds(SMALL, 8)], xsem)

    pltpu.async_copy(noise_hbm.at[pl.ds(chunk_start(0), CH)],
                     noise_v.at[0], nsem[0])
    pltpu.async_copy(noise_hbm.at[pl.ds(chunk_start(1), CH)],
                     noise_v.at[1], nsem[1])
    pltpu.async_copy(mean_hbm, mean_v, msem)
    pltpu.async_copy(lstd_hbm, lstd_v, msem)

    @pl.when(is_big)
    def _():
        pltpu.async_copy(noise_hbm.at[pl.ds(tail_start, TAIL_BIG)],
                         tnoise_v.at[pl.ds(0, TAIL_BIG)], tsem)

    @pl.when(jnp.logical_not(is_big))
    def _():
        pltpu.async_copy(noise_hbm.at[pl.ds(tail_start, TAIL_SMALL)],
                         tnoise_v.at[pl.ds(0, TAIL_SMALL)], tsem)

    pltpu.make_async_copy(ordering_hbm.at[pl.ds(0, CH)], idx0_v,
                          x0sem).wait()
    pltpu.async_copy(table_hbm.at[idx0_v], rows_v.at[0], gsem[0])
    pltpu.make_async_copy(ordering_hbm.at[pl.ds(0, SMALL)],
                          idx_all.at[pl.ds(0, SMALL)], xsem).wait()

    @pl.when(is_big)
    def _():
        pltpu.make_async_copy(ordering_hbm.at[pl.ds(0, 8)],
                              idx_all.at[pl.ds(SMALL, 8)], xsem).wait()

    pltpu.async_copy(table_hbm.at[idx_all.at[pl.ds(CH, CH)]], rows_v.at[1],
                     gsem[1])
    TOFF = NCHUNK * CH

    @pl.when(is_big)
    def _():
        pltpu.async_copy(table_hbm.at[idx_all.at[pl.ds(TOFF, TAIL_BIG)]],
                         trows_v.at[pl.ds(0, TAIL_BIG)], tsem)

    @pl.when(jnp.logical_not(is_big))
    def _():
        pltpu.async_copy(table_hbm.at[idx_all.at[pl.ds(TOFF, TAIL_SMALL)]],
                         trows_v.at[pl.ds(0, TAIL_SMALL)], tsem)

    pltpu.make_async_copy(mean_hbm, mean_v, msem).wait()
    pltpu.make_async_copy(lstd_hbm, lstd_v, msem).wait()
    mean_vals = [mean_v[pl.ds(j * LANES, LANES)] for j in range(D // LANES)]
    scale_vals = [jnp.exp(lstd_v[pl.ds(j * LANES, LANES)])
                  for j in range(D // LANES)]

    def issue_fetch(c, buf):
        pltpu.async_copy(
            table_hbm.at[idx_all.at[pl.ds(c * CH, CH)]], rows_v.at[buf],
            gsem[buf])
        pltpu.async_copy(noise_hbm.at[pl.ds(chunk_start(c), CH)],
                         noise_v.at[buf], nsem[buf])

    def wait_fetch(buf):
        pltpu.make_async_copy(table_hbm.at[idx_all.at[pl.ds(0, CH)]],
                              rows_v.at[buf], gsem[buf]).wait()
        pltpu.make_async_copy(noise_hbm.at[pl.ds(0, CH)], noise_v.at[buf],
                              nsem[buf]).wait()

    def wait_out(buf):
        pltpu.make_async_copy(rows_v.at[buf], out_hbm.at[pl.ds(0, CH)],
                              osem[buf]).wait()

    def compute_rows(buf, nrows):
        def row_body(h, _):
            r = 2 * h
            for dr in range(2):
                for j in range(D // LANES):
                    sl = pl.ds(j * LANES, LANES)
                    rows_v[buf, r + dr, sl] = (
                        rows_v[buf, r + dr, sl] + mean_vals[j]
                        + scale_vals[j] * noise_v[buf, r + dr, sl])
            return None
        lax.fori_loop(0, nrows // 2, row_body, None)

    @pl.loop(0, NT)
    def _triple(t):
        c0 = NB * t
        for i in range(NB):
            b = i
            b2 = (i + 2) % NB
            c = c0 + i
            wait_fetch(b)
            if i == 0:
                @pl.when(t > 0)
                def _():
                    wait_out(b2)
                issue_fetch(c + 2, b2)
            else:
                @pl.when(t < NT - 1)
                def _():
                    wait_out(b2)
                    issue_fetch(c + 2, b2)
            compute_rows(b, CH)
            pltpu.async_copy(rows_v.at[b],
                             out_hbm.at[pl.ds(chunk_start(c), CH)], osem[b])

    def tail(nrows):
        pltpu.make_async_copy(table_hbm.at[idx_all.at[pl.ds(0, nrows)]],
                              trows_v.at[pl.ds(0, nrows)], tsem).wait()
        pltpu.make_async_copy(noise_hbm.at[pl.ds(0, nrows)],
                              tnoise_v.at[pl.ds(0, nrows)], tsem).wait()

        def row_body(r, _):
            for j in range(D // LANES):
                sl = pl.ds(j * LANES, LANES)
                trows_v[r, sl] = (trows_v[r, sl] + mean_vals[j]
                                  + scale_vals[j] * tnoise_v[r, sl])
            return None
        lax.fori_loop(0, nrows, row_body, None)
        pltpu.async_copy(trows_v.at[pl.ds(0, nrows)],
                         out_hbm.at[pl.ds(tail_start, nrows)], tosem)
        pltpu.make_async_copy(trows_v.at[pl.ds(0, nrows)],
                              out_hbm.at[pl.ds(0, nrows)], tosem).wait()

    @pl.when(is_big)
    def _():
        tail(TAIL_BIG)

    @pl.when(jnp.logical_not(is_big))
    def _():
        tail(TAIL_SMALL)

    wait_out(0)
    wait_out(1)
    wait_out(2)


@jax.jit
def _run(ordering, noise, pos_embed, rand_mean, rand_log_std):
    mesh = plsc.VectorSubcoreMesh(core_axis_name="c", subcore_axis_name="s",
                                  num_cores=NC, num_subcores=NS)
    f = pl.kernel(
        _body,
        out_type=jax.ShapeDtypeStruct((N, D), jnp.float32),
        mesh=mesh,
        scratch_types=[
            pltpu.VMEM((BIG,), jnp.int32),
            pltpu.VMEM((CH,), jnp.int32),
            pltpu.VMEM((NB, CH, D), jnp.float32),
            pltpu.VMEM((NB, CH, D), jnp.float32),
            pltpu.VMEM((TAIL_BIG, D), jnp.float32),
            pltpu.VMEM((TAIL_BIG, D), jnp.float32),
            pltpu.VMEM((D,), jnp.float32),
            pltpu.VMEM((D,), jnp.float32),
        ] + [pltpu.SemaphoreType.DMA] * 14,
    )
    return f(ordering, noise, pos_embed, rand_mean, rand_log_std)


def kernel(ordering, noise, pos_embed, rand_mean, rand_log_std):
    return _run(ordering.astype(jnp.int32), noise, pos_embed,
                rand_mean, rand_log_std)
